# speedup vs baseline: 1.3160x; 1.3160x over previous
import functools

import jax
import jax.numpy as jnp
from jax import lax
from jax.experimental import pallas as pl
from jax.experimental.pallas import tpu as pltpu
from jax.experimental.pallas import tpu_sc as plsc

D = 768
B = 4 * 8192
NW = 32
RPW = B // NW
C = 32
NCHUNK = RPW // C
LANES = 16

_mesh = plsc.VectorSubcoreMesh(core_axis_name="c", subcore_axis_name="s")


@functools.partial(
    pl.kernel,
    mesh=_mesh,
    out_type=jax.ShapeDtypeStruct((B, D), jnp.float32),
    scratch_types=[
        pltpu.VMEM((NCHUNK, C), jnp.int32),
        pltpu.VMEM((C, D), jnp.float32),
        pltpu.VMEM((C, D), jnp.float32),
        pltpu.SemaphoreType.DMA,
        pltpu.SemaphoreType.DMA,
    ],
)
def _prepare_encoder(src_hbm, pos_hbm, tab_hbm, out_hbm,
                     idx_v, src_v, gath_v, sem_s, sem_g):
    wid = lax.axis_index("s") * 2 + lax.axis_index("c")
    base = wid * RPW
    pltpu.sync_copy(pos_hbm.at[wid], idx_v)

    def chunk(g, carry):
        rb = base + g * C
        cp_s = pltpu.async_copy(src_hbm.at[pl.ds(rb, C)], src_v, sem_s)
        cp_g = pltpu.async_copy(tab_hbm.at[idx_v.at[g]], gath_v, sem_g)
        cp_s.wait()
        cp_g.wait()

        def row(r, c2):
            for c in range(D // LANES):
                plsc.addupdate(src_v.at[r, pl.ds(c * LANES, LANES)],
                               gath_v[r, pl.ds(c * LANES, LANES)])
            return c2

        lax.fori_loop(0, C, row, 0)
        pltpu.sync_copy(src_v, out_hbm.at[pl.ds(rb, C)])
        return carry

    lax.fori_loop(0, NCHUNK, chunk, 0)


def kernel(src_word, src_pos, emb_table):
    src_flat = src_word.reshape(B, D)
    pos = src_pos.reshape(NW, NCHUNK, C)
    out = _prepare_encoder(src_flat, pos, emb_table)
    return out.reshape(src_word.shape)


# speedup vs baseline: 1.3177x; 1.0013x over previous
import functools

import jax
import jax.numpy as jnp
from jax import lax
from jax.experimental import pallas as pl
from jax.experimental.pallas import tpu as pltpu
from jax.experimental.pallas import tpu_sc as plsc

D = 768
B = 4 * 8192
NW = 32
RPW = B // NW
C = 16
NCHUNK = RPW // C
LANES = 16

_mesh = plsc.VectorSubcoreMesh(core_axis_name="c", subcore_axis_name="s")


@functools.partial(
    pl.kernel,
    mesh=_mesh,
    out_type=jax.ShapeDtypeStruct((B, D), jnp.float32),
    scratch_types=[
        pltpu.VMEM((NCHUNK, C), jnp.int32),
        pltpu.VMEM((2, C, D), jnp.float32),
        pltpu.VMEM((2, C, D), jnp.float32),
        pltpu.VMEM((2, C, D), jnp.float32),
        pltpu.SemaphoreType.DMA,
        pltpu.SemaphoreType.DMA,
        pltpu.SemaphoreType.DMA,
        pltpu.SemaphoreType.DMA,
        pltpu.SemaphoreType.DMA,
        pltpu.SemaphoreType.DMA,
    ],
)
def _prepare_encoder(src_hbm, pos_hbm, tab_hbm, out_hbm,
                     idx_v, src_v, gath_v, out_v,
                     sem_s0, sem_s1, sem_g0, sem_g1, sem_o0, sem_o1):
    sem_s = (sem_s0, sem_s1)
    sem_g = (sem_g0, sem_g1)
    sem_o = (sem_o0, sem_o1)
    wid = lax.axis_index("s") * 2 + lax.axis_index("c")
    base = wid * RPW
    pltpu.sync_copy(pos_hbm.at[wid], idx_v)

    def start_in(g, b):
        rb = base + g * C
        pltpu.async_copy(src_hbm.at[pl.ds(rb, C)], src_v.at[b], sem_s[b])
        pltpu.async_copy(tab_hbm.at[idx_v.at[g]], gath_v.at[b], sem_g[b])

    def wait_in(g, b):
        rb = base + g * C
        pltpu.make_async_copy(src_hbm.at[pl.ds(rb, C)], src_v.at[b],
                              sem_s[b]).wait()
        pltpu.make_async_copy(tab_hbm.at[idx_v.at[g]], gath_v.at[b],
                              sem_g[b]).wait()

    def start_out(g, b):
        rb = base + g * C
        pltpu.async_copy(out_v.at[b], out_hbm.at[pl.ds(rb, C)], sem_o[b])

    def wait_out(g, b):
        rb = base + g * C
        pltpu.make_async_copy(out_v.at[b], out_hbm.at[pl.ds(rb, C)],
                              sem_o[b]).wait()

    def compute(b):
        def row(r, cc):
            for c in range(D // LANES):
                sl = pl.ds(c * LANES, LANES)
                out_v[b, r, sl] = src_v[b, r, sl] + gath_v[b, r, sl]
            return cc
        lax.fori_loop(0, C, row, 0)

    start_in(0, 0)
    start_in(1, 1)
    wait_in(0, 0)
    compute(0)
    start_out(0, 0)
    start_in(2, 0)
    wait_in(1, 1)
    compute(1)
    start_out(1, 1)

    def pair(i, carry):
        for bb in range(2):
            g = 2 * i + bb

            @pl.when(g + 1 < NCHUNK)
            def _():
                start_in(g + 1, 1 - bb)

            wait_in(g, bb)
            wait_out(g - 2, bb)
            compute(bb)
            start_out(g, bb)
        return carry

    lax.fori_loop(1, NCHUNK // 2, pair, 0)
    wait_out(NCHUNK - 2, 0)
    wait_out(NCHUNK - 1, 1)


def kernel(src_word, src_pos, emb_table):
    src_flat = src_word.reshape(B, D)
    pos = src_pos.reshape(NW, NCHUNK, C)
    out = _prepare_encoder(src_flat, pos, emb_table)
    return out.reshape(src_word.shape)


# speedup vs baseline: 1.6632x; 1.2622x over previous
import functools

import jax
import jax.numpy as jnp
from jax import lax
from jax.experimental import pallas as pl
from jax.experimental.pallas import tpu as pltpu
from jax.experimental.pallas import tpu_sc as plsc

D = 768
B = 4 * 8192
NW = 32
RPW = B // NW
C = 16
NCHUNK = RPW // C
NB = 4
LANES = 16

_mesh = plsc.VectorSubcoreMesh(core_axis_name="c", subcore_axis_name="s")


@functools.partial(
    pl.kernel,
    mesh=_mesh,
    out_type=jax.ShapeDtypeStruct((B, D), jnp.float32),
    scratch_types=[
        pltpu.VMEM((NCHUNK, C), jnp.int32),
        pltpu.VMEM((NB, C, D), jnp.float32),
        pltpu.VMEM((NB, C, D), jnp.float32),
    ] + [pltpu.SemaphoreType.DMA] * (3 * NB),
)
def _prepare_encoder(src_hbm, pos_hbm, tab_hbm, out_hbm,
                     idx_v, src_v, gath_v, *sems):
    sem_s = sems[0:NB]
    sem_g = sems[NB:2 * NB]
    sem_o = sems[2 * NB:3 * NB]
    wid = lax.axis_index("s") * 2 + lax.axis_index("c")
    base = wid * RPW
    pltpu.sync_copy(pos_hbm.at[wid], idx_v)

    def start_in(g, b):
        rb = base + g * C
        pltpu.async_copy(src_hbm.at[pl.ds(rb, C)], src_v.at[b], sem_s[b])
        pltpu.async_copy(tab_hbm.at[idx_v.at[g]], gath_v.at[b], sem_g[b])

    def wait_in(g, b):
        rb = base + g * C
        pltpu.make_async_copy(src_hbm.at[pl.ds(rb, C)], src_v.at[b],
                              sem_s[b]).wait()
        pltpu.make_async_copy(tab_hbm.at[idx_v.at[g]], gath_v.at[b],
                              sem_g[b]).wait()

    def start_out(g, b):
        rb = base + g * C
        pltpu.async_copy(gath_v.at[b], out_hbm.at[pl.ds(rb, C)], sem_o[b])

    def wait_out(g, b):
        rb = base + g * C
        pltpu.make_async_copy(gath_v.at[b], out_hbm.at[pl.ds(rb, C)],
                              sem_o[b]).wait()

    def compute(b):
        def row(r, cc):
            for c in range(D // LANES):
                sl = pl.ds(c * LANES, LANES)
                plsc.addupdate(gath_v.at[b, r, sl], src_v[b, r, sl])
            return cc
        lax.fori_loop(0, C, row, 0)

    for g0 in range(NB):
        start_in(g0, g0)

    def quad(i, carry):
        for bb in range(NB):
            g = NB * i + bb
            b2 = (bb + 2) % NB

            @pl.when(jnp.logical_and(g >= 2, g + 2 < NCHUNK))
            def _():
                wait_out(g - 2, b2)
                start_in(g + 2, b2)

            wait_in(g, bb)
            compute(bb)
            start_out(g, bb)
        return carry

    lax.fori_loop(0, NCHUNK // NB, quad, 0)
    for k in range(NB):
        g = NCHUNK - NB + k
        wait_out(g, g % NB)


def kernel(src_word, src_pos, emb_table):
    src_flat = src_word.reshape(B, D)
    pos = src_pos.reshape(NW, NCHUNK, C)
    out = _prepare_encoder(src_flat, pos, emb_table)
    return out.reshape(src_word.shape)


# speedup vs baseline: 2.0259x; 1.2181x over previous
import functools

import jax
import jax.numpy as jnp
from jax import lax
from jax.experimental import pallas as pl
from jax.experimental.pallas import tpu as pltpu
from jax.experimental.pallas import tpu_sc as plsc

D = 768
B = 4 * 8192
NW = 32
RPW = B // NW
C = 8
NCHUNK = RPW // C
NB = 8
LANES = 16

_mesh = plsc.VectorSubcoreMesh(core_axis_name="c", subcore_axis_name="s")


@functools.partial(
    pl.kernel,
    mesh=_mesh,
    out_type=jax.ShapeDtypeStruct((B, D), jnp.float32),
    scratch_types=[
        pltpu.VMEM((NCHUNK, C), jnp.int32),
        pltpu.VMEM((NB, C, D), jnp.float32),
        pltpu.VMEM((NB, C, D), jnp.float32),
    ] + [pltpu.SemaphoreType.DMA] * (3 * NB),
)
def _prepare_encoder(src_hbm, pos_hbm, tab_hbm, out_hbm,
                     idx_v, src_v, gath_v, *sems):
    sem_s = sems[0:NB]
    sem_g = sems[NB:2 * NB]
    sem_o = sems[2 * NB:3 * NB]
    wid = lax.axis_index("s") * 2 + lax.axis_index("c")
    base = wid * RPW
    pltpu.sync_copy(pos_hbm.at[wid], idx_v)

    def start_in(g, b):
        rb = base + g * C
        pltpu.async_copy(src_hbm.at[pl.ds(rb, C)], src_v.at[b], sem_s[b])
        pltpu.async_copy(tab_hbm.at[idx_v.at[g]], gath_v.at[b], sem_g[b])

    def wait_in(g, b):
        rb = base + g * C
        pltpu.make_async_copy(src_hbm.at[pl.ds(rb, C)], src_v.at[b],
                              sem_s[b]).wait()
        pltpu.make_async_copy(tab_hbm.at[idx_v.at[g]], gath_v.at[b],
                              sem_g[b]).wait()

    def start_out(g, b):
        rb = base + g * C
        pltpu.async_copy(gath_v.at[b], out_hbm.at[pl.ds(rb, C)], sem_o[b])

    def wait_out(g, b):
        rb = base + g * C
        pltpu.make_async_copy(gath_v.at[b], out_hbm.at[pl.ds(rb, C)],
                              sem_o[b]).wait()

    def compute(b):
        def row(r, cc):
            for c in range(D // LANES):
                sl = pl.ds(c * LANES, LANES)
                plsc.addupdate(gath_v.at[b, r, sl], src_v[b, r, sl])
            return cc
        lax.fori_loop(0, C, row, 0)

    for g0 in range(NB):
        start_in(g0, g0)

    def quad(i, carry):
        for bb in range(NB):
            g = NB * i + bb
            b2 = (bb + 4) % NB

            @pl.when(jnp.logical_and(g >= 4, g + 4 < NCHUNK))
            def _():
                wait_out(g - 4, b2)
                start_in(g + 4, b2)

            wait_in(g, bb)
            compute(bb)
            start_out(g, bb)
        return carry

    lax.fori_loop(0, NCHUNK // NB, quad, 0)
    for k in range(NB):
        g = NCHUNK - NB + k
        wait_out(g, g % NB)


def kernel(src_word, src_pos, emb_table):
    src_flat = src_word.reshape(B, D)
    pos = src_pos.reshape(NW, NCHUNK, C)
    out = _prepare_encoder(src_flat, pos, emb_table)
    return out.reshape(src_word.shape)
